# Initial kernel scaffold; baseline (speedup 1.0000x reference)
import jax
import jax.numpy as jnp
from jax.experimental import pallas as pl


def kernel(day_of_week, hour_of_day, is_holiday, day_table, hour_table, holiday_table):
    raise NotImplementedError("write your pallas kernel here")


# speedup vs baseline: 10.0032x; 10.0032x over previous
import functools

import jax
import jax.numpy as jnp
from jax import lax
from jax.experimental import pallas as pl
from jax.experimental.pallas import tpu as pltpu
from jax.experimental.pallas import tpu_sc as plsc

EMBED = 64
N_DAY, N_HOUR, N_HOL = 7, 24, 2
N_COMB = N_DAY * N_HOUR * N_HOL

_NC = 2
_NS = 16
_NW = _NC * _NS
_LANES = 16

CHUNK = 1024
GROUP = 128
NGROUP = CHUNK // GROUP


def _ctable_body(day_ref, hour_ref, hol_ref, out_ref):
    hol = hol_ref[:]
    for d in range(N_DAY):
        for h in range(N_HOUR):
            out_ref[pl.ds(d * (N_HOUR * N_HOL) + h * N_HOL, N_HOL), :] = (
                day_ref[pl.ds(d, 1), :] + hour_ref[pl.ds(h, 1), :] + hol
            )


def _build_ctable(day_table, hour_table, holiday_table):
    return pl.pallas_call(
        _ctable_body,
        out_shape=jax.ShapeDtypeStruct((N_COMB, EMBED), jnp.float32),
    )(day_table, hour_table, holiday_table)


def _make_sc_gather(n_rows):
    rows_per_w = n_rows // _NW
    n_chunks = rows_per_w // CHUNK
    mesh = plsc.VectorSubcoreMesh(core_axis_name="c", subcore_axis_name="s")

    @functools.partial(
        pl.kernel,
        mesh=mesh,
        compiler_params=pltpu.CompilerParams(use_tc_tiling_on_sc=False),
        out_type=jax.ShapeDtypeStruct((n_rows, EMBED), jnp.float32),
        scratch_types=[
            pltpu.VMEM((CHUNK,), jnp.int32),
            pltpu.VMEM((CHUNK,), jnp.int32),
            pltpu.VMEM((CHUNK,), jnp.int32),
            pltpu.VMEM((NGROUP, GROUP), jnp.int32),
            pltpu.VMEM((CHUNK, EMBED), jnp.float32),
            pltpu.SemaphoreType.DMA,
        ],
    )
    def sc_kernel(ctable_hbm, day_hbm, hour_hbm, hol_hbm, out_hbm,
                  day_v, hour_v, hol_v, cidx_v, rows_v, sem):
        wid = lax.axis_index("s") * _NC + lax.axis_index("c")
        base = wid * rows_per_w

        def body(t, carry):
            off = base + t * CHUNK
            pltpu.sync_copy(day_hbm.at[pl.ds(off, CHUNK)], day_v)
            pltpu.sync_copy(hour_hbm.at[pl.ds(off, CHUNK)], hour_v)
            pltpu.sync_copy(hol_hbm.at[pl.ds(off, CHUNK)], hol_v)
            for i in range(CHUNK // _LANES):
                s = pl.ds(i * _LANES, _LANES)
                fused = (day_v[s] * (N_HOUR * N_HOL)
                         + hour_v[s] * N_HOL + hol_v[s])
                cidx_v[i // (GROUP // _LANES),
                       pl.ds((i % (GROUP // _LANES)) * _LANES, _LANES)] = fused
            descs = [
                pltpu.async_copy(ctable_hbm.at[cidx_v.at[j]],
                                 rows_v.at[pl.ds(j * GROUP, GROUP)], sem)
                for j in range(NGROUP)
            ]
            for dsc in descs:
                dsc.wait()
            pltpu.sync_copy(rows_v, out_hbm.at[pl.ds(off, CHUNK)])
            return carry

        lax.fori_loop(0, n_chunks, body, 0)

    return sc_kernel


def kernel(day_of_week, hour_of_day, is_holiday, day_table, hour_table,
           holiday_table):
    b, l = day_of_week.shape
    n_rows = b * l
    dow = day_of_week.reshape(n_rows).astype(jnp.int32)
    hod = hour_of_day.reshape(n_rows).astype(jnp.int32)
    hol = is_holiday.reshape(n_rows).astype(jnp.int32)
    ctable = _build_ctable(day_table, hour_table, holiday_table)
    out_flat = _make_sc_gather(n_rows)(ctable, dow, hod, hol)
    return out_flat.reshape(b, l, EMBED)


# speedup vs baseline: 10.0054x; 1.0002x over previous
import functools

import jax
import jax.numpy as jnp
from jax import lax
from jax.experimental import pallas as pl
from jax.experimental.pallas import tpu as pltpu
from jax.experimental.pallas import tpu_sc as plsc

EMBED = 64
N_DAY, N_HOUR, N_HOL = 7, 24, 2
N_COMB = N_DAY * N_HOUR * N_HOL

_NC = 2
_NS = 16
_NW = _NC * _NS
_LANES = 16

CHUNK = 640
GROUP = 128
NGROUP = CHUNK // GROUP


def _ctable_body(day_ref, hour_ref, hol_ref, out_ref):
    hol = hol_ref[:]
    for d in range(N_DAY):
        for h in range(N_HOUR):
            out_ref[pl.ds(d * (N_HOUR * N_HOL) + h * N_HOL, N_HOL), :] = (
                day_ref[pl.ds(d, 1), :] + hour_ref[pl.ds(h, 1), :] + hol
            )


def _build_ctable(day_table, hour_table, holiday_table):
    return pl.pallas_call(
        _ctable_body,
        out_shape=jax.ShapeDtypeStruct((N_COMB, EMBED), jnp.float32),
    )(day_table, hour_table, holiday_table)


def _make_sc_gather(n_rows):
    rows_per_w = n_rows // _NW
    n_chunks = rows_per_w // CHUNK
    mesh = plsc.VectorSubcoreMesh(core_axis_name="c", subcore_axis_name="s")

    @functools.partial(
        pl.kernel,
        mesh=mesh,
        compiler_params=pltpu.CompilerParams(use_tc_tiling_on_sc=False),
        out_type=jax.ShapeDtypeStruct((n_rows, EMBED), jnp.float32),
        scratch_types=[
            pltpu.VMEM((2, CHUNK), jnp.int32),
            pltpu.VMEM((2, CHUNK), jnp.int32),
            pltpu.VMEM((2, CHUNK), jnp.int32),
            pltpu.VMEM((2 * NGROUP, GROUP), jnp.int32),
            pltpu.VMEM((2 * CHUNK, EMBED), jnp.float32),
            pltpu.SemaphoreType.DMA,
            pltpu.SemaphoreType.DMA,
            pltpu.SemaphoreType.DMA,
            pltpu.SemaphoreType.DMA,
            pltpu.SemaphoreType.DMA,
            pltpu.SemaphoreType.DMA,
        ],
    )
    def sc_kernel(ctable_hbm, day_hbm, hour_hbm, hol_hbm, out_hbm,
                  day_v, hour_v, hol_v, cidx_v, rows_v,
                  isem0, isem1, gsem0, gsem1, osem0, osem1):
        isems = (isem0, isem1)
        gsems = (gsem0, gsem1)
        osems = (osem0, osem1)
        wid = lax.axis_index("s") * _NC + lax.axis_index("c")
        base = wid * rows_per_w
        last_off = n_rows - CHUNK

        def issue_idx(t, b):
            off = jnp.minimum(base + t * CHUNK, last_off)
            s = pl.ds(off, CHUNK)
            pltpu.async_copy(day_hbm.at[s], day_v.at[b], isems[b])
            pltpu.async_copy(hour_hbm.at[s], hour_v.at[b], isems[b])
            pltpu.async_copy(hol_hbm.at[s], hol_v.at[b], isems[b])

        def wait_idx(b):
            s = pl.ds(0, CHUNK)
            for ref in (day_v, hour_v, hol_v):
                pltpu.make_async_copy(day_hbm.at[s], ref.at[b], isems[b]).wait()

        def compute_cidx(b):
            for i in range(CHUNK // _LANES):
                s = pl.ds(i * _LANES, _LANES)
                fused = (day_v[b, s] * (N_HOUR * N_HOL)
                         + hour_v[b, s] * N_HOL + hol_v[b, s])
                cidx_v[b * NGROUP + i // (GROUP // _LANES),
                       pl.ds((i % (GROUP // _LANES)) * _LANES, _LANES)] = fused

        def issue_gathers(b):
            for j in range(NGROUP):
                pltpu.async_copy(
                    ctable_hbm.at[cidx_v.at[b * NGROUP + j]],
                    rows_v.at[pl.ds((b * NGROUP + j) * GROUP, GROUP)],
                    gsems[b])

        def wait_gathers(b):
            for j in range(NGROUP):
                pltpu.make_async_copy(
                    ctable_hbm.at[cidx_v.at[b * NGROUP + j]],
                    rows_v.at[pl.ds((b * NGROUP + j) * GROUP, GROUP)],
                    gsems[b]).wait()

        def issue_out(t, b):
            off = base + t * CHUNK
            pltpu.async_copy(rows_v.at[pl.ds(b * CHUNK, CHUNK)],
                             out_hbm.at[pl.ds(off, CHUNK)], osems[b])

        def wait_out(b):
            pltpu.make_async_copy(rows_v.at[pl.ds(b * CHUNK, CHUNK)],
                                  out_hbm.at[pl.ds(base, CHUNK)],
                                  osems[b]).wait()

        issue_idx(0, 0)
        issue_idx(1, 1)
        issue_out(0, 0)
        issue_out(1, 1)

        def chunk_step(t, b):
            wait_idx(b)
            compute_cidx(b)
            issue_idx(t + 2, b)
            wait_out(b)
            issue_gathers(b)
            wait_gathers(b)
            issue_out(t, b)

        def body(k, carry):
            chunk_step(2 * k, 0)
            chunk_step(2 * k + 1, 1)
            return carry

        lax.fori_loop(0, n_chunks // 2, body, 0)
        wait_idx(0)
        wait_idx(1)
        wait_out(0)
        wait_out(1)

    return sc_kernel


def kernel(day_of_week, hour_of_day, is_holiday, day_table, hour_table,
           holiday_table):
    b, l = day_of_week.shape
    n_rows = b * l
    dow = day_of_week.reshape(n_rows).astype(jnp.int32)
    hod = hour_of_day.reshape(n_rows).astype(jnp.int32)
    hol = is_holiday.reshape(n_rows).astype(jnp.int32)
    ctable = _build_ctable(day_table, hour_table, holiday_table)
    out_flat = _make_sc_gather(n_rows)(ctable, dow, hod, hol)
    return out_flat.reshape(b, l, EMBED)
